# baseline (speedup 1.0000x reference)
import dataclasses
import functools

import jax
import jax.numpy as jnp
import numpy as np
from jax.experimental import pallas as pl
from jax.experimental.pallas import tpu as pltpu
from jax.experimental.pallas import tpu_sc as plsc

_K = 6
_SCORE_T = np.float32(0.5)
_TH0 = np.float32(2.5)
_TH1 = np.float32(1.0)
_TH2 = np.float32(1.75)
_C99 = np.float32(0.99)
_C01 = np.float32(0.01)


def _nms_body(scores_ref, at_ref, trajs_ref, sk_ref, fidx_ref):
    _, N, P = scores_ref.shape
    TD = trajs_ref.shape[3]
    scores = scores_ref[0]
    x = trajs_ref[0]
    lx = x[:, :, TD - 4]
    ly = x[:, :, TD - 3]
    at = at_ref[0]

    s_norm = scores / jnp.sum(scores, axis=-1, keepdims=True)
    thresh = (at[:, 0] * _TH0 + at[:, 1] * _TH1 + at[:, 2] * _TH2)[:, None]

    iota_p = jax.lax.broadcasted_iota(jnp.int32, (N, P), 1)
    sc = s_norm
    sks, idxs = [], []
    for _ in range(_K):
        m = jnp.max(sc, axis=-1, keepdims=True)
        idx = jnp.min(jnp.where(sc == m, iota_p, P), axis=-1, keepdims=True)
        oh = (iota_p == idx).astype(jnp.float32)
        ex = jnp.sum(oh * lx, axis=-1, keepdims=True)
        ey = jnp.sum(oh * ly, axis=-1, keepdims=True)
        d0 = lx - ex
        d1 = ly - ey
        within = (jnp.sqrt(d0 * d0 + d1 * d1) < thresh).astype(jnp.float32)
        mask = (np.float32(1.0) - within) * _C99 + _C01
        sc = sc * mask
        sc = jnp.where(oh > 0, np.float32(-1.0), sc)
        sks.append(jnp.sum(oh * s_norm, axis=-1))
        idxs.append(idx[:, 0])

    sk = jnp.stack(sks, axis=-1)
    sk = sk / jnp.sum(sk, axis=-1, keepdims=True)
    logits = jnp.log(sk) / _SCORE_T
    e = jnp.exp(logits - jnp.max(logits, axis=-1, keepdims=True))
    sk_ref[0] = e / jnp.sum(e, axis=-1, keepdims=True)

    row0 = (pl.program_id(0) * N + jax.lax.broadcasted_iota(jnp.int32, (N, _K), 0)) * P
    fidx_ref[0] = row0 + jnp.stack(idxs, axis=-1)


def _sc_gather_t(trajs4, idx):
    S, A, P, TD = trajs4.shape
    T, D = TD // 4, 4
    B = idx.shape[0]
    info = plsc.get_sparse_core_info()
    nc = info.num_cores
    nl = info.num_lanes
    nw = nc * info.num_subcores
    bw = B // nw
    assert nw == S and bw == A * _K and P & (P - 1) == 0
    pbits = P.bit_length() - 1
    mesh = plsc.VectorSubcoreMesh(core_axis_name="c", subcore_axis_name="s")
    n_xy = A * _K * 2
    n_w = A * _K

    @functools.partial(
        pl.kernel,
        mesh=mesh,
        out_type=[
            jax.ShapeDtypeStruct((S, T, n_xy), trajs4.dtype),
            jax.ShapeDtypeStruct((S, T, n_w), trajs4.dtype),
            jax.ShapeDtypeStruct((S, T, n_w), trajs4.dtype),
        ],
        compiler_params=dataclasses.replace(
            pltpu.CompilerParams(use_tc_tiling_on_sc=False),
            needs_layout_passes=False,
        ),
        scratch_types=[
            pltpu.VMEM((bw,), jnp.int32),
            pltpu.VMEM((bw, TD), trajs4.dtype),
            pltpu.VMEM((n_xy,), trajs4.dtype),
            pltpu.VMEM((n_w,), trajs4.dtype),
            pltpu.VMEM((n_w,), trajs4.dtype),
            pltpu.SemaphoreType.DMA,
        ],
    )
    def gk(trajs_hbm, idx_hbm, xy_hbm, yaw_hbm, spd_hbm,
           idx_s, rows_v, xy_v, yaw_v, spd_v, sem):
        wid = jax.lax.axis_index("s") * nc + jax.lax.axis_index("c")
        pltpu.sync_copy(idx_hbm.at[pl.ds(wid * bw, bw)], idx_s)

        @pl.loop(0, bw // nl)
        def _(grp):
            vec = idx_s[pl.ds(grp * nl, nl)]
            for i in range(nl):
                f = vec[i]
                j = grp * nl + i
                a = (f >> pbits) & (A - 1)
                p = f & (P - 1)
                pltpu.async_copy(
                    trajs_hbm.at[wid, a, p], rows_v.at[j], sem
                )

        @pl.loop(0, bw)
        def _(j):
            pltpu.make_async_copy(
                trajs_hbm.at[wid, 0, 0], rows_v.at[j], sem
            ).wait()

        lane = jax.lax.iota(jnp.int32, nl)

        @pl.loop(0, T)
        def _(t):
            c0 = t * D

            @pl.loop(0, n_xy // nl)
            def _(g):
                j = g * nl + lane
                r = j >> 1
                c = c0 + (j & 1)
                xy_v[pl.ds(g * nl, nl)] = plsc.load_gather(rows_v, [r, c])

            @pl.loop(0, n_w // nl)
            def _(g):
                r = g * nl + lane
                yaw_v[pl.ds(g * nl, nl)] = plsc.load_gather(
                    rows_v, [r, jnp.full((nl,), c0 + 2, jnp.int32)])
                spd_v[pl.ds(g * nl, nl)] = plsc.load_gather(
                    rows_v, [r, jnp.full((nl,), c0 + 3, jnp.int32)])

            pltpu.sync_copy(xy_v, xy_hbm.at[wid, t])
            pltpu.sync_copy(yaw_v, yaw_hbm.at[wid, t])
            pltpu.sync_copy(spd_v, spd_hbm.at[wid, t])

    return gk(trajs4, idx)


def kernel(valid, scores, trajs, agent_type):
    S, A, P, T, D = trajs.shape
    N = S * A
    TD = T * D

    trajs4 = trajs.reshape(S, A, P, TD)

    sk, fidx = pl.pallas_call(
        _nms_body,
        grid=(S,),
        in_specs=[
            pl.BlockSpec((1, A, P), lambda i: (i, 0, 0)),
            pl.BlockSpec((1, A, 3), lambda i: (i, 0, 0)),
            pl.BlockSpec((1, A, P, TD), lambda i: (i, 0, 0, 0)),
        ],
        out_specs=[
            pl.BlockSpec((1, A, _K), lambda i: (i, 0, 0)),
            pl.BlockSpec((1, A, _K), lambda i: (i, 0, 0)),
        ],
        out_shape=[
            jax.ShapeDtypeStruct((S, A, _K), jnp.float32),
            jax.ShapeDtypeStruct((S, A, _K), jnp.int32),
        ],
        compiler_params=pltpu.CompilerParams(
            dimension_semantics=("parallel",),
        ),
    )(scores, agent_type, trajs4)

    xy, yaw, spd = _sc_gather_t(trajs4, fidx.reshape(N * _K))
    waymo_trajs = xy.reshape(S, T, A, _K, 2)
    waymo_yaw = yaw.reshape(S, T, A, _K, 1)
    waymo_spd = spd.reshape(S, T, A, _K, 1)
    waymo_valid = jnp.broadcast_to(valid[:, None, :], (S, T, A))
    return (waymo_valid, waymo_trajs, sk.reshape(S, A, _K), waymo_yaw, waymo_spd)


# speedup vs baseline: 1.0024x; 1.0024x over previous
import dataclasses
import functools

import jax
import jax.numpy as jnp
import numpy as np
from jax.experimental import pallas as pl
from jax.experimental.pallas import tpu as pltpu
from jax.experimental.pallas import tpu_sc as plsc

_K = 6
_SCORE_T = np.float32(0.5)
_TH0 = np.float32(2.5)
_TH1 = np.float32(1.0)
_TH2 = np.float32(1.75)
_C99 = np.float32(0.99)
_C01 = np.float32(0.01)


def _nms_body(scores_ref, at_ref, tail_ref, sk_ref, fidx_ref):
    _, N, P = scores_ref.shape
    W = tail_ref.shape[3]
    scores = scores_ref[0]
    x = tail_ref[0]
    del W
    lx = x[:, :, 60]
    ly = x[:, :, 61]
    at = at_ref[0]

    s_norm = scores / jnp.sum(scores, axis=-1, keepdims=True)
    thresh = (at[:, 0] * _TH0 + at[:, 1] * _TH1 + at[:, 2] * _TH2)[:, None]

    iota_p = jax.lax.broadcasted_iota(jnp.int32, (N, P), 1)
    sc = s_norm
    sks, idxs = [], []
    for _ in range(_K):
        m = jnp.max(sc, axis=-1, keepdims=True)
        idx = jnp.min(jnp.where(sc == m, iota_p, P), axis=-1, keepdims=True)
        oh = (iota_p == idx).astype(jnp.float32)
        ex = jnp.sum(oh * lx, axis=-1, keepdims=True)
        ey = jnp.sum(oh * ly, axis=-1, keepdims=True)
        d0 = lx - ex
        d1 = ly - ey
        within = (jnp.sqrt(d0 * d0 + d1 * d1) < thresh).astype(jnp.float32)
        mask = (np.float32(1.0) - within) * _C99 + _C01
        sc = sc * mask
        sc = jnp.where(oh > 0, np.float32(-1.0), sc)
        sks.append(jnp.sum(oh * s_norm, axis=-1))
        idxs.append(idx[:, 0])

    sk = jnp.stack(sks, axis=-1)
    sk = sk / jnp.sum(sk, axis=-1, keepdims=True)
    logits = jnp.log(sk) / _SCORE_T
    e = jnp.exp(logits - jnp.max(logits, axis=-1, keepdims=True))
    sk_ref[0] = e / jnp.sum(e, axis=-1, keepdims=True)

    row0 = (pl.program_id(0) * N + jax.lax.broadcasted_iota(jnp.int32, (N, _K), 0)) * P
    fidx_ref[0] = row0 + jnp.stack(idxs, axis=-1)


def _sc_gather_t(trajs4, idx):
    S, A, P, TD = trajs4.shape
    T, D = TD // 4, 4
    B = idx.shape[0]
    info = plsc.get_sparse_core_info()
    nc = info.num_cores
    nl = info.num_lanes
    nw = nc * info.num_subcores
    bw = B // nw
    assert nw == S and bw == A * _K and P & (P - 1) == 0
    pbits = P.bit_length() - 1
    mesh = plsc.VectorSubcoreMesh(core_axis_name="c", subcore_axis_name="s")
    n_xy = A * _K * 2
    n_w = A * _K

    @functools.partial(
        pl.kernel,
        mesh=mesh,
        out_type=[
            jax.ShapeDtypeStruct((S, T, n_xy), trajs4.dtype),
            jax.ShapeDtypeStruct((S, T, n_w), trajs4.dtype),
            jax.ShapeDtypeStruct((S, T, n_w), trajs4.dtype),
        ],
        compiler_params=dataclasses.replace(
            pltpu.CompilerParams(use_tc_tiling_on_sc=False),
            needs_layout_passes=False,
        ),
        scratch_types=[
            pltpu.VMEM((bw,), jnp.int32),
            pltpu.VMEM((bw, TD), trajs4.dtype),
            pltpu.VMEM((n_xy,), trajs4.dtype),
            pltpu.VMEM((n_w,), trajs4.dtype),
            pltpu.VMEM((n_w,), trajs4.dtype),
            pltpu.SemaphoreType.DMA,
        ],
    )
    def gk(trajs_hbm, idx_hbm, xy_hbm, yaw_hbm, spd_hbm,
           idx_s, rows_v, xy_v, yaw_v, spd_v, sem):
        wid = jax.lax.axis_index("s") * nc + jax.lax.axis_index("c")
        pltpu.sync_copy(idx_hbm.at[pl.ds(wid * bw, bw)], idx_s)

        @pl.loop(0, bw // nl)
        def _(grp):
            vec = idx_s[pl.ds(grp * nl, nl)]
            for i in range(nl):
                f = vec[i]
                j = grp * nl + i
                a = (f >> pbits) & (A - 1)
                p = f & (P - 1)
                pltpu.async_copy(
                    trajs_hbm.at[wid, a, p], rows_v.at[j], sem
                )

        @pl.loop(0, bw)
        def _(j):
            pltpu.make_async_copy(
                trajs_hbm.at[wid, 0, 0], rows_v.at[j], sem
            ).wait()

        lane = jax.lax.iota(jnp.int32, nl)

        @pl.loop(0, T)
        def _(t):
            c0 = t * D

            @pl.loop(0, n_xy // nl)
            def _(g):
                j = g * nl + lane
                r = j >> 1
                c = c0 + (j & 1)
                xy_v[pl.ds(g * nl, nl)] = plsc.load_gather(rows_v, [r, c])

            @pl.loop(0, n_w // nl)
            def _(g):
                r = g * nl + lane
                yaw_v[pl.ds(g * nl, nl)] = plsc.load_gather(
                    rows_v, [r, jnp.full((nl,), c0 + 2, jnp.int32)])
                spd_v[pl.ds(g * nl, nl)] = plsc.load_gather(
                    rows_v, [r, jnp.full((nl,), c0 + 3, jnp.int32)])

            pltpu.sync_copy(xy_v, xy_hbm.at[wid, t])
            pltpu.sync_copy(yaw_v, yaw_hbm.at[wid, t])
            pltpu.sync_copy(spd_v, spd_hbm.at[wid, t])

    return gk(trajs4, idx)


def kernel(valid, scores, trajs, agent_type):
    S, A, P, T, D = trajs.shape
    N = S * A
    TD = T * D

    trajs4 = trajs.reshape(S, A, P, TD)

    sk, fidx = pl.pallas_call(
        _nms_body,
        grid=(S,),
        in_specs=[
            pl.BlockSpec((1, A, P), lambda i: (i, 0, 0)),
            pl.BlockSpec((1, A, 3), lambda i: (i, 0, 0)),
            pl.BlockSpec((1, A, P, 128), lambda i: (i, 0, 0, 2)),
        ],
        out_specs=[
            pl.BlockSpec((1, A, _K), lambda i: (i, 0, 0)),
            pl.BlockSpec((1, A, _K), lambda i: (i, 0, 0)),
        ],
        out_shape=[
            jax.ShapeDtypeStruct((S, A, _K), jnp.float32),
            jax.ShapeDtypeStruct((S, A, _K), jnp.int32),
        ],
        compiler_params=pltpu.CompilerParams(
            dimension_semantics=("parallel",),
        ),
    )(scores, agent_type, trajs4)

    xy, yaw, spd = _sc_gather_t(trajs4, fidx.reshape(N * _K))
    waymo_trajs = xy.reshape(S, T, A, _K, 2)
    waymo_yaw = yaw.reshape(S, T, A, _K, 1)
    waymo_spd = spd.reshape(S, T, A, _K, 1)
    waymo_valid = jnp.broadcast_to(valid[:, None, :], (S, T, A))
    return (waymo_valid, waymo_trajs, sk.reshape(S, A, _K), waymo_yaw, waymo_spd)


# speedup vs baseline: 1.6543x; 1.6503x over previous
import dataclasses
import functools

import jax
import jax.numpy as jnp
import numpy as np
from jax.experimental import pallas as pl
from jax.experimental.pallas import tpu as pltpu
from jax.experimental.pallas import tpu_sc as plsc

_K = 6
_SCORE_T = np.float32(0.5)
_TH0 = np.float32(2.5)
_TH1 = np.float32(1.0)
_TH2 = np.float32(1.75)
_C99 = np.float32(0.99)
_C01 = np.float32(0.01)


def _nms_body(scores_ref, at_ref, last_ref, sk_ref, fidx_ref):
    N, P = scores_ref.shape
    scores = scores_ref[...]
    lx = last_ref[:, 0, :]
    ly = last_ref[:, 1, :]
    at = at_ref[...]

    s_norm = scores / jnp.sum(scores, axis=-1, keepdims=True)
    thresh = (at[:, 0] * _TH0 + at[:, 1] * _TH1 + at[:, 2] * _TH2)[:, None]

    iota_p = jax.lax.broadcasted_iota(jnp.int32, (N, P), 1)
    sc = s_norm
    sks, idxs = [], []
    for _ in range(_K):
        m = jnp.max(sc, axis=-1, keepdims=True)
        idx = jnp.min(jnp.where(sc == m, iota_p, P), axis=-1, keepdims=True)
        oh = (iota_p == idx).astype(jnp.float32)
        ex = jnp.sum(oh * lx, axis=-1, keepdims=True)
        ey = jnp.sum(oh * ly, axis=-1, keepdims=True)
        d0 = lx - ex
        d1 = ly - ey
        within = (jnp.sqrt(d0 * d0 + d1 * d1) < thresh).astype(jnp.float32)
        mask = (np.float32(1.0) - within) * _C99 + _C01
        sc = sc * mask
        sc = jnp.where(oh > 0, np.float32(-1.0), sc)
        sks.append(jnp.sum(oh * s_norm, axis=-1))
        idxs.append(idx[:, 0])

    sk = jnp.stack(sks, axis=-1)
    sk = sk / jnp.sum(sk, axis=-1, keepdims=True)
    logits = jnp.log(sk) / _SCORE_T
    e = jnp.exp(logits - jnp.max(logits, axis=-1, keepdims=True))
    sk_ref[...] = e / jnp.sum(e, axis=-1, keepdims=True)

    row0 = jax.lax.broadcasted_iota(jnp.int32, (N, _K), 0) * P
    fidx_ref[...] = row0 + jnp.stack(idxs, axis=-1)


def _sc_gather_t(trajs4, idx):
    S, A, P, TD = trajs4.shape
    T, D = TD // 4, 4
    B = idx.shape[0]
    info = plsc.get_sparse_core_info()
    nc = info.num_cores
    nl = info.num_lanes
    nw = nc * info.num_subcores
    bw = B // nw
    assert nw == S and bw == A * _K and P & (P - 1) == 0
    pbits = P.bit_length() - 1
    mesh = plsc.VectorSubcoreMesh(core_axis_name="c", subcore_axis_name="s")
    n_xy = A * _K * 2
    n_w = A * _K

    @functools.partial(
        pl.kernel,
        mesh=mesh,
        out_type=[
            jax.ShapeDtypeStruct((S, T, n_xy), trajs4.dtype),
            jax.ShapeDtypeStruct((S, T, n_w), trajs4.dtype),
            jax.ShapeDtypeStruct((S, T, n_w), trajs4.dtype),
        ],
        compiler_params=dataclasses.replace(
            pltpu.CompilerParams(use_tc_tiling_on_sc=False),
            needs_layout_passes=False,
        ),
        scratch_types=[
            pltpu.VMEM((bw,), jnp.int32),
            pltpu.VMEM((bw, TD), trajs4.dtype),
            pltpu.VMEM((n_xy,), trajs4.dtype),
            pltpu.VMEM((n_w,), trajs4.dtype),
            pltpu.VMEM((n_w,), trajs4.dtype),
            pltpu.SemaphoreType.DMA,
        ],
    )
    def gk(trajs_hbm, idx_hbm, xy_hbm, yaw_hbm, spd_hbm,
           idx_s, rows_v, xy_v, yaw_v, spd_v, sem):
        wid = jax.lax.axis_index("s") * nc + jax.lax.axis_index("c")
        pltpu.sync_copy(idx_hbm.at[pl.ds(wid * bw, bw)], idx_s)

        @pl.loop(0, bw // nl)
        def _(grp):
            vec = idx_s[pl.ds(grp * nl, nl)]
            for i in range(nl):
                f = vec[i]
                j = grp * nl + i
                a = (f >> pbits) & (A - 1)
                p = f & (P - 1)
                pltpu.async_copy(
                    trajs_hbm.at[wid, a, p], rows_v.at[j], sem
                )

        @pl.loop(0, bw)
        def _(j):
            pltpu.make_async_copy(
                trajs_hbm.at[wid, 0, 0], rows_v.at[j], sem
            ).wait()

        lane = jax.lax.iota(jnp.int32, nl)

        @pl.loop(0, T)
        def _(t):
            c0 = t * D

            @pl.loop(0, n_xy // nl)
            def _(g):
                j = g * nl + lane
                r = j >> 1
                c = c0 + (j & 1)
                xy_v[pl.ds(g * nl, nl)] = plsc.load_gather(rows_v, [r, c])

            @pl.loop(0, n_w // nl)
            def _(g):
                r = g * nl + lane
                yaw_v[pl.ds(g * nl, nl)] = plsc.load_gather(
                    rows_v, [r, jnp.full((nl,), c0 + 2, jnp.int32)])
                spd_v[pl.ds(g * nl, nl)] = plsc.load_gather(
                    rows_v, [r, jnp.full((nl,), c0 + 3, jnp.int32)])

            pltpu.sync_copy(xy_v, xy_hbm.at[wid, t])
            pltpu.sync_copy(yaw_v, yaw_hbm.at[wid, t])
            pltpu.sync_copy(spd_v, spd_hbm.at[wid, t])

    return gk(trajs4, idx)


def kernel(valid, scores, trajs, agent_type):
    S, A, P, T, D = trajs.shape
    N = S * A
    TD = T * D

    trajs4 = trajs.reshape(S, A, P, TD)
    last = jnp.moveaxis(trajs[:, :, :, T - 1, :2], -1, 2).reshape(N, 2, P)

    sk, fidx = pl.pallas_call(
        _nms_body,
        in_specs=[
            pl.BlockSpec((N, P), lambda: (0, 0)),
            pl.BlockSpec((N, 3), lambda: (0, 0)),
            pl.BlockSpec((N, 2, P), lambda: (0, 0, 0)),
        ],
        out_specs=[
            pl.BlockSpec((N, _K), lambda: (0, 0)),
            pl.BlockSpec((N, _K), lambda: (0, 0)),
        ],
        out_shape=[
            jax.ShapeDtypeStruct((N, _K), jnp.float32),
            jax.ShapeDtypeStruct((N, _K), jnp.int32),
        ],
    )(scores.reshape(N, P), agent_type.reshape(N, 3), last)

    xy, yaw, spd = _sc_gather_t(trajs4, fidx.reshape(N * _K))
    waymo_trajs = xy.reshape(S, T, A, _K, 2)
    waymo_yaw = yaw.reshape(S, T, A, _K, 1)
    waymo_spd = spd.reshape(S, T, A, _K, 1)
    waymo_valid = jnp.broadcast_to(valid[:, None, :], (S, T, A))
    return (waymo_valid, waymo_trajs, sk.reshape(S, A, _K), waymo_yaw, waymo_spd)
